# Initial kernel scaffold; baseline (speedup 1.0000x reference)
import jax
import jax.numpy as jnp
from jax.experimental import pallas as pl


def kernel(x, edge_index, edge_weight, W1, b1, W2, b2):
    raise NotImplementedError("write your pallas kernel here")


# speedup vs baseline: 9.2047x; 9.2047x over previous
import functools

import jax
import jax.numpy as jnp
from jax import lax
from jax.experimental import pallas as pl
from jax.experimental.pallas import tpu as pltpu
from jax.experimental.pallas import tpu_sc as plsc

N = 10000
E = 320000
D_IN = 128
D_H = 128
D_OUT = 40
D_OUT_PAD = 64

NC = 2
NS = 16
NW = NC * NS
EPW = E // NW
CHUNK = 80
NCHUNKS = EPW // CHUNK
ROWS_PER_INIT = N // 10

_MESH = plsc.VectorSubcoreMesh(core_axis_name="c", subcore_axis_name="s")


def _wid():
    return lax.axis_index("s") * NC + lax.axis_index("c")


DEG_SLAB = 2000


@functools.partial(
    pl.kernel,
    out_type=jax.ShapeDtypeStruct((NC * N,), jnp.float32),
    mesh=_MESH,
    scratch_types=[
        pltpu.VMEM_SHARED((N,), jnp.float32),
        pltpu.VMEM((CHUNK,), jnp.int32),
        pltpu.VMEM((CHUNK,), jnp.float32),
        pltpu.VMEM((DEG_SLAB,), jnp.float32),
        pltpu.SemaphoreType.DMA,
    ],
)
def _sc_deg(dst_hbm, ew_hbm, out_hbm, acc_sh, dst_v, ew_v, slab_v, sem):
    cid = lax.axis_index("c")
    sid = lax.axis_index("s")
    wid = _wid()

    @pl.when(sid < 5)
    def _init():
        z16 = jnp.zeros((16,), jnp.float32)

        def zfill(i, _):
            slab_v[pl.ds(i * 16, 16)] = z16
            return 0

        lax.fori_loop(0, DEG_SLAB // 16, zfill, 0)
        pltpu.sync_copy(slab_v, acc_sh.at[pl.ds(sid * DEG_SLAB, DEG_SLAB)])

    plsc.subcore_barrier()

    def step(ci, _):
        base = pl.multiple_of(wid * EPW + ci * CHUNK, 8)
        pltpu.sync_copy(dst_hbm.at[pl.ds(base, CHUNK)], dst_v)
        pltpu.sync_copy(ew_hbm.at[pl.ds(base, CHUNK)], ew_v)
        pltpu.sync_copy(ew_v, acc_sh.at[dst_v], add=True)
        return 0

    lax.fori_loop(0, NCHUNKS, step, 0)
    plsc.subcore_barrier()

    @pl.when(sid < 5)
    def _dump():
        r0 = sid * DEG_SLAB
        pltpu.sync_copy(acc_sh.at[pl.ds(r0, DEG_SLAB)], slab_v)
        pltpu.sync_copy(slab_v, out_hbm.at[pl.ds(cid * N + r0, DEG_SLAB)])


SLAB_ROWS = 200


def _make_sc_agg(d):
    @functools.partial(
        pl.kernel,
        out_type=jax.ShapeDtypeStruct((NC, N, d), jnp.float32),
        mesh=_MESH,
        scratch_types=[
            pltpu.VMEM_SHARED((N, d), jnp.float32),
            pltpu.VMEM((CHUNK,), jnp.int32),
            pltpu.VMEM((CHUNK,), jnp.int32),
            pltpu.VMEM((CHUNK,), jnp.float32),
            pltpu.VMEM((CHUNK, d), jnp.float32),
            pltpu.VMEM((SLAB_ROWS, d), jnp.float32),
            pltpu.SemaphoreType.DMA,
        ],
        compiler_params=pltpu.CompilerParams(use_tc_tiling_on_sc=False),
    )
    def _sc_agg(g_hbm, src_hbm, dst_hbm, ew_hbm, out_hbm,
                acc_sh, src_v, dst_v, ew_v, rows_v, slab_v, sem):
        cid = lax.axis_index("c")
        sid = lax.axis_index("s")
        wid = _wid()

        @pl.when(sid < 10)
        def _init():
            z16 = jnp.zeros((16,), jnp.float32)

            def zfill(i, _):
                for j in range(d // 16):
                    slab_v[i, pl.ds(j * 16, 16)] = z16
                return 0

            lax.fori_loop(0, SLAB_ROWS, zfill, 0)
            r0 = sid * ROWS_PER_INIT
            for k in range(ROWS_PER_INIT // SLAB_ROWS):
                pltpu.sync_copy(
                    slab_v, acc_sh.at[pl.ds(r0 + k * SLAB_ROWS, SLAB_ROWS)])

        plsc.subcore_barrier()

        def step(ci, _):
            base = pl.multiple_of(wid * EPW + ci * CHUNK, 8)
            pltpu.sync_copy(src_hbm.at[pl.ds(base, CHUNK)], src_v)
            pltpu.sync_copy(ew_hbm.at[pl.ds(base, CHUNK)], ew_v)
            pltpu.async_copy(g_hbm.at[src_v], rows_v, sem).wait()

            def scale_grp(gi, _):
                ew16 = ew_v[pl.ds(gi * 16, 16)]
                for l in range(16):
                    s = ew16[l]
                    i = gi * 16 + l
                    for j in range(d // 16):
                        sl = pl.ds(j * 16, 16)
                        rows_v[i, sl] = rows_v[i, sl] * s
                return 0

            lax.fori_loop(0, CHUNK // 16, scale_grp, 0)
            pltpu.sync_copy(dst_hbm.at[pl.ds(base, CHUNK)], dst_v)
            pltpu.sync_copy(rows_v, acc_sh.at[dst_v], add=True)
            return 0

        lax.fori_loop(0, NCHUNKS, step, 0)
        plsc.subcore_barrier()

        @pl.when(sid < 10)
        def _dump():
            r0 = sid * ROWS_PER_INIT
            for k in range(ROWS_PER_INIT // SLAB_ROWS):
                rk = r0 + k * SLAB_ROWS
                pltpu.sync_copy(acc_sh.at[pl.ds(rk, SLAB_ROWS)], slab_v)
                pltpu.sync_copy(slab_v, out_hbm.at[cid, pl.ds(rk, SLAB_ROWS)])

    return _sc_agg


_sc_agg_h = _make_sc_agg(D_H)
_sc_agg_o = _make_sc_agg(D_OUT_PAD)


def _dis_from(degp):
    deg = degp[0] + degp[1] + 1.0
    return lax.rsqrt(deg)


def _tc1_body(degp_ref, x_ref, w1_ref, g1_ref):
    dis = _dis_from(degp_ref[...])
    h = jnp.dot(x_ref[...], w1_ref[...], preferred_element_type=jnp.float32)
    g1_ref[...] = h * dis[:, None]


def _tc2_body(degp_ref, aggp_ref, g1_ref, w2_ref, b1_ref, g2_ref):
    dis = _dis_from(degp_ref[...])
    acc = aggp_ref[0] + aggp_ref[1] + g1_ref[...]
    out1 = acc * dis[:, None] + b1_ref[...][None, :]
    h = jnp.maximum(out1, 0.0)
    h2 = jnp.dot(h, w2_ref[...], preferred_element_type=jnp.float32)
    g2_ref[...] = h2 * dis[:, None]


def _tc3_body(degp_ref, aggp_ref, g2_ref, b2_ref, out_ref):
    dis = _dis_from(degp_ref[...])
    acc = aggp_ref[0] + aggp_ref[1] + g2_ref[...]
    out_ref[...] = acc * dis[:, None] + b2_ref[...][None, :]


def _tc1(degp, x, W1):
    return pl.pallas_call(
        _tc1_body,
        out_shape=jax.ShapeDtypeStruct((N, D_H), jnp.float32),
    )(degp, x, W1)


def _tc2(degp, aggp, g1, W2p, b1):
    return pl.pallas_call(
        _tc2_body,
        out_shape=jax.ShapeDtypeStruct((N, D_OUT_PAD), jnp.float32),
    )(degp, aggp, g1, W2p, b1)


def _tc3(degp, aggp, g2, b2p):
    return pl.pallas_call(
        _tc3_body,
        out_shape=jax.ShapeDtypeStruct((N, D_OUT_PAD), jnp.float32),
    )(degp, aggp, g2, b2p)


def kernel(x, edge_index, edge_weight, W1, b1, W2, b2):
    src = edge_index[0]
    dst = edge_index[1]
    W2p = jnp.pad(W2, ((0, 0), (0, D_OUT_PAD - D_OUT)))
    b2p = jnp.pad(b2, (0, D_OUT_PAD - D_OUT))

    degp = _sc_deg(dst, edge_weight).reshape(NC, N)
    g1 = _tc1(degp, x, W1)
    aggp1 = _sc_agg_h(g1, src, dst, edge_weight)
    g2 = _tc2(degp, aggp1, g1, W2p, b1)
    aggp2 = _sc_agg_o(g2, src, dst, edge_weight)
    out = _tc3(degp, aggp2, g2, b2p)
    return out[:, :D_OUT]
